# Initial kernel scaffold; baseline (speedup 1.0000x reference)
import jax
import jax.numpy as jnp
from jax.experimental import pallas as pl


def kernel(x, indexes, labels, weights):
    raise NotImplementedError("write your pallas kernel here")


# speedup vs baseline: 1.2227x; 1.2227x over previous
import functools

import jax
import jax.numpy as jnp
from jax import lax
from jax.experimental import pallas as pl
from jax.experimental.pallas import tpu as pltpu
from jax.experimental.pallas import tpu_sc as plsc

B = 1024
N = 100000
Q = 0.7
K = 0.5

CB = 2048
NB = (N + CB - 1) // CB

NC, NS, L = 2, 16, 16
NW = NC * NS
BPW = B // NW


@functools.lru_cache(maxsize=None)
def _sc_gather_build():
    mesh = plsc.VectorSubcoreMesh(core_axis_name="c", subcore_axis_name="s")

    @functools.partial(
        pl.kernel,
        mesh=mesh,
        out_type=[
            jax.ShapeDtypeStruct((B,), jnp.int32),
            jax.ShapeDtypeStruct((B,), jnp.float32),
            jax.ShapeDtypeStruct((B,), jnp.float32),
        ],
        scratch_types=[
            pltpu.VMEM((BPW,), jnp.int32),
            pltpu.VMEM((BPW,), jnp.int32),
            pltpu.VMEM((BPW,), jnp.int32),
            pltpu.VMEM((BPW,), jnp.float32),
            pltpu.VMEM((BPW,), jnp.float32),
            pltpu.VMEM((BPW,), jnp.float32),
            pltpu.SemaphoreType.DMA,
        ],
    )
    def sc_gather(idx_hbm, lab_hbm, w_hbm, xf_hbm, y_out, ed_out, wb_out,
                  idx_v, flat_v, y_v, w_v, xd_v, ed_v, sem):
        wid = lax.axis_index("s") * NC + lax.axis_index("c")
        base = wid * BPW
        pltpu.sync_copy(idx_hbm.at[pl.ds(base, BPW)], idx_v)
        pltpu.async_copy(lab_hbm.at[idx_v], y_v, sem).wait()
        pltpu.async_copy(w_hbm.at[idx_v], w_v, sem).wait()
        for c in range(BPW // L):
            sl = pl.ds(c * L, L)
            rows = lax.iota(jnp.int32, L) + (base + c * L)
            flat_v[sl] = rows * N + idx_v[sl]
        pltpu.async_copy(xf_hbm.at[flat_v], xd_v, sem).wait()
        for c in range(BPW // L):
            sl = pl.ds(c * L, L)
            ed_v[sl] = jnp.exp(xd_v[sl])
        pltpu.sync_copy(y_v, y_out.at[pl.ds(base, BPW)])
        pltpu.sync_copy(ed_v, ed_out.at[pl.ds(base, BPW)])
        pltpu.sync_copy(w_v, wb_out.at[pl.ds(base, BPW)])

    return sc_gather


def _tc_body(xb, labb, y, ed, wb, out, p_acc, z_acc):
    i = pl.program_id(0)

    @pl.when(i == 0)
    def _init():
        p_acc[...] = jnp.zeros_like(p_acc)
        z_acc[...] = jnp.zeros_like(z_acc)

    e = jnp.exp(xb[...])
    col = i * CB + lax.broadcasted_iota(jnp.int32, (1, CB), 1)
    e = jnp.where(col < N, e, 0.0)
    m = labb[0] == y[...]
    z_acc[...] += jnp.sum(e, axis=1, keepdims=True)
    p_acc[...] += jnp.sum(jnp.where(m, e, 0.0), axis=1, keepdims=True)

    @pl.when(i == NB - 1)
    def _fin():
        edv = ed[...]
        p = p_acc[...] - edv
        z = z_acc[...] - edv
        prob = p / z
        a = (1.0 - prob ** Q) / Q
        mean_w = jnp.mean(wb[...])
        out[0, 0] = jnp.mean(a) * mean_w - ((1.0 - K ** Q) / Q) * mean_w


_tc_call = pl.pallas_call(
    _tc_body,
    grid=(NB,),
    in_specs=[
        pl.BlockSpec((B, CB), lambda i: (0, i)),
        pl.BlockSpec((1, 1, CB), lambda i: (i, 0, 0)),
        pl.BlockSpec((B, 1), lambda i: (0, 0)),
        pl.BlockSpec((B, 1), lambda i: (0, 0)),
        pl.BlockSpec((B, 1), lambda i: (0, 0)),
    ],
    out_specs=pl.BlockSpec(memory_space=pltpu.SMEM),
    out_shape=jax.ShapeDtypeStruct((1, 1), jnp.float32),
    scratch_shapes=[
        pltpu.VMEM((B, 1), jnp.float32),
        pltpu.VMEM((B, 1), jnp.float32),
    ],
    compiler_params=pltpu.CompilerParams(
        dimension_semantics=("arbitrary",),
    ),
)


def kernel(x, indexes, labels, weights):
    idx = indexes.astype(jnp.int32)
    lab = labels.astype(jnp.int32)
    y, ed, wb = _sc_gather_build()(idx, lab, weights.reshape(-1),
                                   x.reshape(-1))
    labp = jnp.pad(lab, (0, NB * CB - N)).reshape(NB, 1, CB)
    loss = _tc_call(x, labp, y.reshape(B, 1), ed.reshape(B, 1),
                    wb.reshape(B, 1))
    return loss[0, 0]


# speedup vs baseline: 2.5256x; 2.0655x over previous
import functools

import jax
import jax.numpy as jnp
from jax import lax
from jax.experimental import pallas as pl
from jax.experimental.pallas import tpu as pltpu
from jax.experimental.pallas import tpu_sc as plsc

B = 1024
N = 100000
Q = 0.7
K = 0.5

CB = 2048
NB = (N + CB - 1) // CB

NC, NS, L = 2, 16, 16
NW = NC * NS
BPW = B // NW


@functools.lru_cache(maxsize=None)
def _sc_gather_build():
    mesh = plsc.VectorSubcoreMesh(core_axis_name="c", subcore_axis_name="s")

    @functools.partial(
        pl.kernel,
        mesh=mesh,
        out_type=[
            jax.ShapeDtypeStruct((B,), jnp.int32),
            jax.ShapeDtypeStruct((B,), jnp.float32),
        ],
        scratch_types=[
            pltpu.VMEM((BPW,), jnp.int32),
            pltpu.VMEM((BPW,), jnp.int32),
            pltpu.VMEM((BPW,), jnp.float32),
            pltpu.SemaphoreType.DMA,
        ],
    )
    def sc_gather(idx_hbm, lab_hbm, w_hbm, y_out, wb_out,
                  idx_v, y_v, w_v, sem):
        wid = lax.axis_index("s") * NC + lax.axis_index("c")
        base = wid * BPW
        pltpu.sync_copy(idx_hbm.at[pl.ds(base, BPW)], idx_v)
        pltpu.async_copy(lab_hbm.at[idx_v], y_v, sem).wait()
        pltpu.async_copy(w_hbm.at[idx_v], w_v, sem).wait()
        pltpu.sync_copy(y_v, y_out.at[pl.ds(base, BPW)])
        pltpu.sync_copy(w_v, wb_out.at[pl.ds(base, BPW)])

    return sc_gather


def _tc_body(xb, labb, y, idxb, wb, out, p_acc, z_acc):
    i = pl.program_id(0)

    @pl.when(i == 0)
    def _init():
        p_acc[...] = jnp.zeros_like(p_acc)
        z_acc[...] = jnp.zeros_like(z_acc)

    e = jnp.exp(xb[...])
    col = i * CB + lax.broadcasted_iota(jnp.int32, (1, CB), 1)
    keep = (col != idxb[...]) & (col < N)
    e = jnp.where(keep, e, 0.0)
    m = labb[0] == y[...]
    z_acc[...] += jnp.sum(e, axis=1, keepdims=True)
    p_acc[...] += jnp.sum(jnp.where(m, e, 0.0), axis=1, keepdims=True)

    @pl.when(i == NB - 1)
    def _fin():
        prob = p_acc[...] / z_acc[...]
        a = (1.0 - prob ** Q) / Q
        mean_w = jnp.mean(wb[...])
        out[0, 0] = jnp.mean(a) * mean_w - ((1.0 - K ** Q) / Q) * mean_w


_tc_call = pl.pallas_call(
    _tc_body,
    grid=(NB,),
    in_specs=[
        pl.BlockSpec((B, CB), lambda i: (0, i)),
        pl.BlockSpec((1, 1, CB), lambda i: (i, 0, 0)),
        pl.BlockSpec((B, 1), lambda i: (0, 0)),
        pl.BlockSpec((B, 1), lambda i: (0, 0)),
        pl.BlockSpec((B, 1), lambda i: (0, 0)),
    ],
    out_specs=pl.BlockSpec(memory_space=pltpu.SMEM),
    out_shape=jax.ShapeDtypeStruct((1, 1), jnp.float32),
    scratch_shapes=[
        pltpu.VMEM((B, 1), jnp.float32),
        pltpu.VMEM((B, 1), jnp.float32),
    ],
    compiler_params=pltpu.CompilerParams(
        dimension_semantics=("arbitrary",),
    ),
)


def kernel(x, indexes, labels, weights):
    idx = indexes.astype(jnp.int32)
    lab = labels.astype(jnp.int32)
    y, wb = _sc_gather_build()(idx, lab, weights.reshape(-1))
    labp = jnp.pad(lab, (0, NB * CB - N)).reshape(NB, 1, CB)
    loss = _tc_call(x, labp, y.reshape(B, 1), idx.reshape(B, 1),
                    wb.reshape(B, 1))
    return loss[0, 0]


# speedup vs baseline: 2.5645x; 1.0154x over previous
import functools

import jax
import jax.numpy as jnp
from jax import lax
from jax.experimental import pallas as pl
from jax.experimental.pallas import tpu as pltpu
from jax.experimental.pallas import tpu_sc as plsc

B = 1024
N = 100000
Q = 0.7
K = 0.5

CB = 4096
NB = (N + CB - 1) // CB

NC, NS, L = 2, 16, 16
NW = NC * NS
BPW = B // NW


@functools.lru_cache(maxsize=None)
def _sc_gather_build():
    mesh = plsc.VectorSubcoreMesh(core_axis_name="c", subcore_axis_name="s")

    @functools.partial(
        pl.kernel,
        mesh=mesh,
        out_type=[
            jax.ShapeDtypeStruct((B,), jnp.int32),
            jax.ShapeDtypeStruct((B,), jnp.float32),
        ],
        scratch_types=[
            pltpu.VMEM((BPW,), jnp.int32),
            pltpu.VMEM((BPW,), jnp.int32),
            pltpu.VMEM((BPW,), jnp.float32),
            pltpu.SemaphoreType.DMA,
        ],
    )
    def sc_gather(idx_hbm, lab_hbm, w_hbm, y_out, wb_out,
                  idx_v, y_v, w_v, sem):
        wid = lax.axis_index("s") * NC + lax.axis_index("c")
        base = wid * BPW
        pltpu.sync_copy(idx_hbm.at[pl.ds(base, BPW)], idx_v)
        pltpu.async_copy(lab_hbm.at[idx_v], y_v, sem).wait()
        pltpu.async_copy(w_hbm.at[idx_v], w_v, sem).wait()
        pltpu.sync_copy(y_v, y_out.at[pl.ds(base, BPW)])
        pltpu.sync_copy(w_v, wb_out.at[pl.ds(base, BPW)])

    return sc_gather


def _tc_body(xb, labb, y, idxb, wb, out, p_acc, z_acc):
    i = pl.program_id(0)

    @pl.when(i == 0)
    def _init():
        p_acc[...] = jnp.zeros_like(p_acc)
        z_acc[...] = jnp.zeros_like(z_acc)

    e = jnp.exp(xb[...])
    col = i * CB + lax.broadcasted_iota(jnp.int32, (1, CB), 1)
    keep = (col != idxb[...]) & (col < N)
    e = jnp.where(keep, e, 0.0)
    m = labb[0] == y[...]
    z_acc[...] += jnp.sum(e, axis=1, keepdims=True)
    p_acc[...] += jnp.sum(jnp.where(m, e, 0.0), axis=1, keepdims=True)

    @pl.when(i == NB - 1)
    def _fin():
        prob = p_acc[...] / z_acc[...]
        a = (1.0 - prob ** Q) / Q
        mean_w = jnp.mean(wb[...])
        out[0, 0] = jnp.mean(a) * mean_w - ((1.0 - K ** Q) / Q) * mean_w


_tc_call = pl.pallas_call(
    _tc_body,
    grid=(NB,),
    in_specs=[
        pl.BlockSpec((B, CB), lambda i: (0, i)),
        pl.BlockSpec((1, 1, CB), lambda i: (i, 0, 0)),
        pl.BlockSpec((B, 1), lambda i: (0, 0)),
        pl.BlockSpec((B, 1), lambda i: (0, 0)),
        pl.BlockSpec((B, 1), lambda i: (0, 0)),
    ],
    out_specs=pl.BlockSpec(memory_space=pltpu.SMEM),
    out_shape=jax.ShapeDtypeStruct((1, 1), jnp.float32),
    scratch_shapes=[
        pltpu.VMEM((B, 1), jnp.float32),
        pltpu.VMEM((B, 1), jnp.float32),
    ],
    compiler_params=pltpu.CompilerParams(
        dimension_semantics=("arbitrary",),
    ),
)


def kernel(x, indexes, labels, weights):
    idx = indexes.astype(jnp.int32)
    lab = labels.astype(jnp.int32)
    y, wb = _sc_gather_build()(idx, lab, weights.reshape(-1))
    labp = jnp.pad(lab, (0, NB * CB - N)).reshape(NB, 1, CB)
    loss = _tc_call(x, labp, y.reshape(B, 1), idx.reshape(B, 1),
                    wb.reshape(B, 1))
    return loss[0, 0]
